# Initial kernel scaffold; baseline (speedup 1.0000x reference)
import jax
import jax.numpy as jnp
from jax.experimental import pallas as pl


def kernel(x_user, x_item, W1_ui, b1_ui, W2_ui, b2_ui, W1_iu, b1_iu, W2_iu, b2_iu, edge_index_user_rates_item, edge_index_item_rev_rates_user):
    raise NotImplementedError("write your pallas kernel here")


# speedup vs baseline: 5.0873x; 5.0873x over previous
import functools

import jax
import jax.numpy as jnp
from jax import lax
from jax.experimental import pallas as pl
from jax.experimental.pallas import tpu as pltpu
from jax.experimental.pallas import tpu_sc as plsc

_CH = 128
_NS = 16


@functools.lru_cache(maxsize=None)
def _build_sc_agg(N, H, prefetch):
    rpt = (N // _NS) // 8 * 8
    last = N - (_NS - 1) * rpt
    mesh = plsc.VectorSubcoreMesh(core_axis_name="c", subcore_axis_name="s")

    @functools.partial(
        pl.kernel,
        mesh=mesh,
        out_type=(jax.ShapeDtypeStruct((2 * N, H), jnp.float32),
                  jax.ShapeDtypeStruct((2 * N, H), jnp.float32)),
        scratch_types=[
            pltpu.VMEM((prefetch, _CH), jnp.int32),
            pltpu.VMEM((prefetch, _CH), jnp.int32),
            pltpu.VMEM((_CH, H), jnp.float32),
            pltpu.VMEM_SHARED((N + 8, H), jnp.float32),
            pltpu.SemaphoreType.DMA,
        ],
        compiler_params=pltpu.CompilerParams(use_tc_tiling_on_sc=False),
    )
    def sc_agg(xu_lo, xu_hi, xi_lo, xi_hi,
               src_ui, dst_ui, src_iu, dst_iu,
               out_lo, out_hi, sidx, didx, rows, acc, sem):
        c = lax.axis_index("c")
        s = lax.axis_index("s")

        def relation(x_src, x_dst, src3d, dst3d, out, out_base):
            r0 = pl.multiple_of(s * rpt, 8)
            @pl.when(s < _NS - 1)
            def _():
                pltpu.sync_copy(x_dst.at[pl.ds(r0, rpt)],
                                acc.at[pl.ds(r0, rpt)])

            @pl.when(s == _NS - 1)
            def _():
                pltpu.sync_copy(x_dst.at[pl.ds(r0, last)],
                                acc.at[pl.ds(r0, last)])

            pltpu.sync_copy(src3d.at[s], sidx)
            pltpu.sync_copy(dst3d.at[s], didx)
            plsc.subcore_barrier()

            def body(i, carry):
                pltpu.async_copy(x_src.at[sidx.at[i]], rows, sem).wait()
                pltpu.sync_copy(rows, acc.at[didx.at[i]], add=True)
                return carry

            lax.fori_loop(0, prefetch, body, 0)
            plsc.subcore_barrier()

            @pl.when(s < _NS - 1)
            def _():
                pltpu.sync_copy(acc.at[pl.ds(r0, rpt)],
                                out.at[pl.ds(out_base + r0, rpt)])

            @pl.when(s == _NS - 1)
            def _():
                pltpu.sync_copy(acc.at[pl.ds(r0, last)],
                                out.at[pl.ds(out_base + r0, last)])

        @pl.when(c == 0)
        def _():
            relation(xi_lo, xu_lo, src_iu, dst_iu, out_lo, 0)
            relation(xu_lo, xi_lo, src_ui, dst_ui, out_lo, N)

        @pl.when(c == 1)
        def _():
            relation(xi_hi, xu_hi, src_iu, dst_iu, out_hi, 0)
            relation(xu_hi, xi_hi, src_ui, dst_ui, out_hi, N)

    return sc_agg


def _mlp_block(hlo_ref, hhi_ref, w1_ref, b1_ref, w2_ref, b2_ref, o_ref):
    H = hlo_ref.shape[-1]
    z = (jnp.dot(hlo_ref[...], w1_ref[0, :H], preferred_element_type=jnp.float32)
         + jnp.dot(hhi_ref[...], w1_ref[0, H:], preferred_element_type=jnp.float32))
    z = jnp.maximum(z + b1_ref[0], 0.0)
    o_ref[...] = (jnp.dot(z, w2_ref[0], preferred_element_type=jnp.float32)
                  + b2_ref[0])


def _mlp(h_lo, h_hi, W1, b1, W2, b2, N, D, H, B):
    nb = N // B
    return pl.pallas_call(
        _mlp_block,
        grid=(2, nb),
        in_specs=[
            pl.BlockSpec((B, H), lambda r, i: (r * nb + i, 0)),
            pl.BlockSpec((B, H), lambda r, i: (r * nb + i, 0)),
            pl.BlockSpec((1, D, D), lambda r, i: (r, 0, 0)),
            pl.BlockSpec((1, 1, D), lambda r, i: (r, 0, 0)),
            pl.BlockSpec((1, D, D), lambda r, i: (r, 0, 0)),
            pl.BlockSpec((1, 1, D), lambda r, i: (r, 0, 0)),
        ],
        out_specs=pl.BlockSpec((B, D), lambda r, i: (r * nb + i, 0)),
        out_shape=jax.ShapeDtypeStruct((2 * N, D), jnp.float32),
    )(h_lo, h_hi, W1, b1, W2, b2)


def kernel(x_user, x_item, W1_ui, b1_ui, W2_ui, b2_ui,
           W1_iu, b1_iu, W2_iu, b2_iu,
           edge_index_user_rates_item, edge_index_item_rev_rates_user):
    N, D = x_user.shape
    H = D // 2
    E = edge_index_user_rates_item.shape[1]

    prefetch = -(-E // (_CH * _NS))
    pad_e = _NS * prefetch * _CH

    def prep(ei):
        src = ei[0].astype(jnp.int32)
        dst = ei[1].astype(jnp.int32)
        if pad_e > E:
            src = jnp.pad(src, (0, pad_e - E))
            dst = jnp.pad(dst, (0, pad_e - E), constant_values=N)
        return (src.reshape(_NS, prefetch, _CH),
                dst.reshape(_NS, prefetch, _CH))

    src_ui2, dst_ui2 = prep(edge_index_user_rates_item)
    src_iu2, dst_iu2 = prep(edge_index_item_rev_rates_user)

    sc_agg = _build_sc_agg(N, H, prefetch)
    h_lo, h_hi = sc_agg(x_user[:, :H], x_user[:, H:],
                        x_item[:, :H], x_item[:, H:],
                        src_ui2, dst_ui2, src_iu2, dst_iu2)

    W1 = jnp.stack([W1_iu, W1_ui])
    b1 = jnp.stack([b1_iu, b1_ui])[:, None, :]
    W2 = jnp.stack([W2_iu, W2_ui])
    b2 = jnp.stack([b2_iu, b2_ui])[:, None, :]
    out = _mlp(h_lo, h_hi, W1, b1, W2, b2, N, D, H, B=1000)
    return out[:N], out[N:]
